# baseline (speedup 1.0000x reference)
import functools

import jax
import jax.numpy as jnp
from jax import lax
from jax.experimental import pallas as pl
from jax.experimental.pallas import tpu as pltpu

_VOCAB = 1000
_VPAD = 1024
_HIDDEN = 128
_BATCH = 4096
_SEQ = 20
_BBLK = 32
_GRID = _BATCH // _BBLK


def _table_body(emb_ref, w_ref, b_ref, out_ref):
    out_ref[...] = (
        lax.dot_general(
            emb_ref[...], w_ref[...],
            (((1,), (1,)), ((), ())),
            preferred_element_type=jnp.float32,
            precision=lax.Precision.HIGHEST,
        ) + b_ref[...]
    ).astype(jnp.bfloat16)


def _compute_table(emb, w, b):
    return pl.pallas_call(
        _table_body,
        out_shape=jax.ShapeDtypeStruct((_VPAD, _VPAD), jnp.bfloat16),
    )(emb, w, b.reshape(1, _VPAD))


def _onehot_body(ids_ref, t_ref, out_ref):
    ids = ids_ref[...]
    vocab = lax.broadcasted_iota(jnp.int32, (_BBLK, _SEQ, _VPAD), 2)
    onehot = (ids[:, :, None] == vocab).astype(jnp.bfloat16)
    logits = lax.dot_general(
        onehot, t_ref[...],
        (((2,), (0,)), ((), ())),
        preferred_element_type=jnp.float32,
    )
    out_ref[...] = logits[:, :, :_VOCAB]


def _onehot_gather(ids, table):
    return pl.pallas_call(
        _onehot_body,
        grid=(_GRID,),
        in_specs=[
            pl.BlockSpec((_BBLK, _SEQ), lambda i: (i, 0)),
            pl.BlockSpec((_VPAD, _VPAD), lambda i: (0, 0)),
        ],
        out_specs=pl.BlockSpec((_BBLK, _SEQ, _VOCAB), lambda i: (i, 0, 0)),
        out_shape=jax.ShapeDtypeStruct((_BATCH, _SEQ, _VOCAB), jnp.float32),
    )(ids, table)


def kernel(input_ids, embed_table, head_w, head_b):
    emb_pad = jnp.pad(embed_table, ((0, _VPAD - _VOCAB), (0, 0)))
    w_pad = jnp.pad(head_w, ((0, _VPAD - _VOCAB), (0, 0)))
    b_pad = jnp.pad(head_b, (0, _VPAD - _VOCAB))
    table = _compute_table(emb_pad, w_pad, b_pad)
    return _onehot_gather(input_ids.astype(jnp.int32), table)


# speedup vs baseline: 1.0169x; 1.0169x over previous
import jax
import jax.numpy as jnp
from jax import lax
from jax.experimental import pallas as pl

_VOCAB = 1000
_VPAD = 1024
_HIDDEN = 128
_BATCH = 4096
_SEQ = 20
_BBLK = 32
_GRID = _BATCH // _BBLK


def _body(ids_ref, emb_ref, wt_ref, b_ref, out_ref):
    ids = ids_ref[...]
    vocab = lax.broadcasted_iota(jnp.int32, (_BBLK, _SEQ, _VPAD), 2)
    onehot = (ids[:, :, None] == vocab).astype(jnp.bfloat16)
    h = lax.dot_general(
        onehot, emb_ref[...],
        (((2,), (0,)), ((), ())),
        preferred_element_type=jnp.float32,
    ).astype(jnp.bfloat16)
    logits = lax.dot_general(
        h, wt_ref[...],
        (((2,), (0,)), ((), ())),
        preferred_element_type=jnp.float32,
    ) + b_ref[...][None]
    out_ref[...] = logits[:, :, :_VOCAB]


def kernel(input_ids, embed_table, head_w, head_b):
    emb = jnp.pad(embed_table, ((0, _VPAD - _VOCAB), (0, 0))).astype(jnp.bfloat16)
    wt = head_w.T.astype(jnp.bfloat16)
    wt = jnp.pad(wt, ((0, 0), (0, _VPAD - _VOCAB)))
    b = jnp.pad(head_b, (0, _VPAD - _VOCAB)).reshape(1, _VPAD)
    return pl.pallas_call(
        _body,
        grid=(_GRID,),
        in_specs=[
            pl.BlockSpec((_BBLK, _SEQ), lambda i: (i, 0)),
            pl.BlockSpec((_VPAD, _HIDDEN), lambda i: (0, 0)),
            pl.BlockSpec((_HIDDEN, _VPAD), lambda i: (0, 0)),
            pl.BlockSpec((1, _VPAD), lambda i: (0, 0)),
        ],
        out_specs=pl.BlockSpec((_BBLK, _SEQ, _VOCAB), lambda i: (i, 0, 0)),
        out_shape=jax.ShapeDtypeStruct((_BATCH, _SEQ, _VOCAB), jnp.float32),
    )(input_ids.astype(jnp.int32), emb, wt, b)


# speedup vs baseline: 1.0186x; 1.0017x over previous
import jax
import jax.numpy as jnp
from jax import lax
from jax.experimental import pallas as pl

_VOCAB = 1000
_VPAD = 1024
_HIDDEN = 128
_BATCH = 4096
_SEQ = 20
_BBLK = 32
_TOK = _BBLK * _SEQ
_GRID = _BATCH // _BBLK


def _body(ids_ref, emb_ref, wt_ref, b_ref, out_ref):
    ids = ids_ref[0]
    vocab = lax.broadcasted_iota(jnp.int32, (_VPAD, _TOK), 0)
    onehot_t = (ids == vocab).astype(jnp.bfloat16)
    h = lax.dot_general(
        onehot_t, emb_ref[...],
        (((0,), (0,)), ((), ())),
        preferred_element_type=jnp.float32,
    ).astype(jnp.bfloat16)
    logits = lax.dot_general(
        h, wt_ref[...],
        (((1,), (0,)), ((), ())),
        preferred_element_type=jnp.float32,
    ) + b_ref[...]
    for c in range(_BBLK):
        out_ref[c] = logits[c * _SEQ : (c + 1) * _SEQ, : _VOCAB]


def kernel(input_ids, embed_table, head_w, head_b):
    ids = input_ids.astype(jnp.int32).reshape(_GRID, 1, _TOK)
    emb = jnp.pad(embed_table, ((0, _VPAD - _VOCAB), (0, 0))).astype(jnp.bfloat16)
    wt = head_w.T.astype(jnp.bfloat16)
    wt = jnp.pad(wt, ((0, 0), (0, _VPAD - _VOCAB)))
    b = jnp.pad(head_b, (0, _VPAD - _VOCAB)).reshape(1, _VPAD)
    return pl.pallas_call(
        _body,
        grid=(_GRID,),
        in_specs=[
            pl.BlockSpec((1, 1, _TOK), lambda i: (i, 0, 0)),
            pl.BlockSpec((_VPAD, _HIDDEN), lambda i: (0, 0)),
            pl.BlockSpec((_HIDDEN, _VPAD), lambda i: (0, 0)),
            pl.BlockSpec((1, _VPAD), lambda i: (0, 0)),
        ],
        out_specs=pl.BlockSpec((_BBLK, _SEQ, _VOCAB), lambda i: (i, 0, 0)),
        out_shape=jax.ShapeDtypeStruct((_BATCH, _SEQ, _VOCAB), jnp.float32),
    )(ids, emb, wt, b)


# speedup vs baseline: 1.0705x; 1.0509x over previous
import jax
import jax.numpy as jnp
from jax import lax
from jax.experimental import pallas as pl

_VOCAB = 1000
_VPAD = 1024
_HIDDEN = 128
_BATCH = 4096
_SEQ = 20
_BBLK = 64
_TOK = _BBLK * _SEQ
_GRID = _BATCH // _BBLK


def _body(ids_ref, emb_ref, wt_ref, b_ref, out_ref):
    ids = ids_ref[0]
    vocab = lax.broadcasted_iota(jnp.int32, (_VPAD, _TOK), 0)
    onehot_t = (ids == vocab).astype(jnp.bfloat16)
    h = lax.dot_general(
        onehot_t, emb_ref[...],
        (((0,), (0,)), ((), ())),
        preferred_element_type=jnp.float32,
    ).astype(jnp.bfloat16)
    logits = lax.dot_general(
        h, wt_ref[...],
        (((1,), (0,)), ((), ())),
        preferred_element_type=jnp.float32,
    ) + b_ref[...]
    for c in range(_BBLK):
        out_ref[c] = logits[c * _SEQ : (c + 1) * _SEQ, : _VOCAB]


def kernel(input_ids, embed_table, head_w, head_b):
    ids = input_ids.astype(jnp.int32).reshape(_GRID, 1, _TOK)
    emb = jnp.pad(embed_table, ((0, _VPAD - _VOCAB), (0, 0))).astype(jnp.bfloat16)
    wt = head_w.T.astype(jnp.bfloat16)
    wt = jnp.pad(wt, ((0, 0), (0, _VPAD - _VOCAB)))
    b = jnp.pad(head_b, (0, _VPAD - _VOCAB)).reshape(1, _VPAD)
    return pl.pallas_call(
        _body,
        grid=(_GRID,),
        in_specs=[
            pl.BlockSpec((1, 1, _TOK), lambda i: (i, 0, 0)),
            pl.BlockSpec((_VPAD, _HIDDEN), lambda i: (0, 0)),
            pl.BlockSpec((_HIDDEN, _VPAD), lambda i: (0, 0)),
            pl.BlockSpec((1, _VPAD), lambda i: (0, 0)),
        ],
        out_specs=pl.BlockSpec((_BBLK, _SEQ, _VOCAB), lambda i: (i, 0, 0)),
        out_shape=jax.ShapeDtypeStruct((_BATCH, _SEQ, _VOCAB), jnp.float32),
    )(ids, emb, wt, b)
